# Initial kernel scaffold; baseline (speedup 1.0000x reference)
import jax
import jax.numpy as jnp
from jax.experimental import pallas as pl


def kernel(x):
    raise NotImplementedError("write your pallas kernel here")


# speedup vs baseline: 17.7780x; 17.7780x over previous
import functools

import jax
import jax.numpy as jnp
from jax.experimental import pallas as pl
from jax.experimental.pallas import tpu as pltpu

_OFFSETS = [(-1, -1), (-1, 0), (-1, 1), (0, 1), (1, 1), (1, 0), (1, -1), (0, -1)]
_R = 8


def _lbp_hist_body(x_ref, o_ref, code_ref, *, H, W):
    r = x_ref[0, 0]
    g = x_ref[0, 1]
    b = x_ref[0, 2]
    gray = 0.2989 * r + 0.587 * g + 0.114 * b

    gp = jnp.concatenate([gray[:1], gray, gray[-1:]], axis=0)
    gp = jnp.concatenate([gp[:, :1], gp, gp[:, -1:]], axis=1)

    code = jnp.zeros((H, W), jnp.int32)
    for k, (dy, dx) in enumerate(_OFFSETS):
        neigh = jax.lax.slice(gp, (1 + dy, 1 + dx), (1 + dy + H, 1 + dx + W))
        code = code + (1 << k) * (neigh >= gray).astype(jnp.int32)
    code_ref[...] = code

    def step(i, acc):
        cc = code_ref[pl.ds(i * _R, _R), :]
        hi = cc >> 4
        lo = cc & 15
        iota = jax.lax.broadcasted_iota(jnp.int32, (_R, 16, W), 1)
        a = (hi[:, None, :] == iota).astype(jnp.bfloat16).reshape(_R * 16, W)
        bm = (lo[:, None, :] == iota).astype(jnp.bfloat16).reshape(_R * 16, W)
        return acc + jax.lax.dot_general(
            a, bm, (((1,), (1,)), ((), ())), preferred_element_type=jnp.float32)

    acc = jax.lax.fori_loop(
        0, H // _R, step, jnp.zeros((_R * 16, _R * 16), jnp.float32))

    m16 = jnp.zeros((16, 16), jnp.float32)
    for rr in range(_R):
        m16 = m16 + acc[rr * 16:(rr + 1) * 16, rr * 16:(rr + 1) * 16]

    norm = jnp.sqrt(jnp.sum(m16 * m16))
    o_ref[0] = m16 / (norm + 1e-6)


def kernel(x):
    bs, _, H, W = x.shape
    body = functools.partial(_lbp_hist_body, H=H, W=W)
    out = pl.pallas_call(
        body,
        grid=(bs,),
        in_specs=[pl.BlockSpec((1, 3, H, W), lambda i: (i, 0, 0, 0))],
        out_specs=pl.BlockSpec((1, 16, 16), lambda i: (i, 0, 0)),
        out_shape=jax.ShapeDtypeStruct((bs, 16, 16), jnp.float32),
        scratch_shapes=[pltpu.VMEM((H, W), jnp.int32)],
    )(x)
    return out.reshape(bs, 256)


# speedup vs baseline: 30.1698x; 1.6970x over previous
import functools

import jax
import jax.numpy as jnp
from jax import lax
from jax.experimental import pallas as pl
from jax.experimental.pallas import tpu as pltpu
from jax.experimental.pallas import tpu_sc as plsc

_OFFSETS = [(-1, -1), (-1, 0), (-1, 1), (0, 1), (1, 1), (1, 0), (1, -1), (0, -1)]

_NC = 2
_NS = 16
_L = 16


def _code_body(x_ref, o_ref, *, H, W):
    r = x_ref[0, 0]
    g = x_ref[0, 1]
    b = x_ref[0, 2]
    gray = 0.2989 * r + 0.587 * g + 0.114 * b

    gp = jnp.concatenate([gray[:1], gray, gray[-1:]], axis=0)
    gp = jnp.concatenate([gp[:, :1], gp, gp[:, -1:]], axis=1)

    code = jnp.zeros((H, W), jnp.int32)
    for k, (dy, dx) in enumerate(_OFFSETS):
        neigh = jax.lax.slice(gp, (1 + dy, 1 + dx), (1 + dy + H, 1 + dx + W))
        code = code + (1 << k) * (neigh >= gray).astype(jnp.int32)
    o_ref[0] = code


def _make_sc_hist(n_img, n_pix, chunk):
    n_chunks = n_pix // chunk
    mesh = plsc.VectorSubcoreMesh(core_axis_name="c", subcore_axis_name="s")

    @functools.partial(
        pl.kernel,
        mesh=mesh,
        compiler_params=pltpu.CompilerParams(needs_layout_passes=False),
        out_type=jax.ShapeDtypeStruct((n_img, _L * 256), jnp.float32),
        scratch_types=[
            pltpu.VMEM((chunk,), jnp.int32),
            pltpu.VMEM((chunk,), jnp.int32),
            pltpu.VMEM((_L * 256,), jnp.float32),
            pltpu.SemaphoreType.DMA,
            pltpu.SemaphoreType.DMA,
        ],
    )
    def sc_hist(codes_hbm, out_hbm, buf0, buf1, hist_v, sem0, sem1):
        wid = lax.axis_index("s") * _NC + lax.axis_index("c")

        def zero_step(i, _):
            hist_v[pl.ds(i * _L, _L)] = jnp.zeros((_L,), jnp.float32)
            return 0
        lax.fori_loop(0, (_L * 256) // _L, zero_step, 0)

        lane256 = lax.broadcasted_iota(jnp.int32, (_L,), 0) * 256
        ones = jnp.ones((_L,), jnp.float32)

        bufs = [buf0, buf1]
        sems = [sem0, sem1]

        def start(c, slot):
            return pltpu.async_copy(
                codes_hbm.at[wid, pl.ds(c * chunk, chunk)], bufs[slot], sems[slot])

        def consume(slot):
            buf = bufs[slot]

            def scat(j, _):
                c = buf[pl.ds(j * _L, _L)]
                plsc.addupdate_scatter(hist_v, [c + lane256], ones)
                return 0
            lax.fori_loop(0, chunk // _L, scat, 0)

        cp = start(0, 0)
        for c in range(n_chunks):
            slot = c % 2
            cp.wait()
            if c + 1 < n_chunks:
                nxt = start(c + 1, 1 - slot)
            consume(slot)
            if c + 1 < n_chunks:
                cp = nxt

        pltpu.sync_copy(hist_v, out_hbm.at[wid])

    return sc_hist


def _reduce_body(h_ref, o_ref):
    h = h_ref[0]
    s = jnp.sum(h, axis=0)
    norm = jnp.sqrt(jnp.sum(s * s))
    o_ref[0, 0] = s / (norm + 1e-6)


def kernel(x):
    bs, _, H, W = x.shape
    n_pix = H * W

    codes = pl.pallas_call(
        functools.partial(_code_body, H=H, W=W),
        grid=(bs,),
        in_specs=[pl.BlockSpec((1, 3, H, W), lambda i: (i, 0, 0, 0))],
        out_specs=pl.BlockSpec((1, H, W), lambda i: (i, 0, 0)),
        out_shape=jax.ShapeDtypeStruct((bs, H, W), jnp.int32),
    )(x)

    chunk = 32768 if n_pix % 32768 == 0 else n_pix
    sc_hist = _make_sc_hist(bs, n_pix, chunk)
    hist_lanes = sc_hist(codes.reshape(bs, n_pix))

    out = pl.pallas_call(
        _reduce_body,
        grid=(bs,),
        in_specs=[pl.BlockSpec((1, _L, 256), lambda i: (i, 0, 0))],
        out_specs=pl.BlockSpec((1, 1, 256), lambda i: (i, 0, 0)),
        out_shape=jax.ShapeDtypeStruct((bs, 1, 256), jnp.float32),
    )(hist_lanes.reshape(bs, _L, 256))
    return out.reshape(bs, 256)


# speedup vs baseline: 33.0406x; 1.0952x over previous
import functools

import jax
import jax.numpy as jnp
from jax import lax
from jax.experimental import pallas as pl
from jax.experimental.pallas import tpu as pltpu
from jax.experimental.pallas import tpu_sc as plsc

_OFFSETS = [(-1, -1), (-1, 0), (-1, 1), (0, 1), (1, 1), (1, 0), (1, -1), (0, -1)]

_NC = 2
_NS = 16
_L = 16


def _code_body(x_ref, o_ref, *, H, W):
    r = x_ref[0, 0]
    g = x_ref[0, 1]
    b = x_ref[0, 2]
    gray = 0.2989 * r + 0.587 * g + 0.114 * b

    gp = jnp.concatenate([gray[:1], gray, gray[-1:]], axis=0)
    gp = jnp.concatenate([gp[:, :1], gp, gp[:, -1:]], axis=1)

    code = jnp.zeros((H, W), jnp.int32)
    for k, (dy, dx) in enumerate(_OFFSETS):
        neigh = jax.lax.slice(gp, (1 + dy, 1 + dx), (1 + dy + H, 1 + dx + W))
        code = code + (1 << k) * (neigh >= gray).astype(jnp.int32)
    o_ref[0] = code


def _make_sc_hist(n_img, H, W, rows):
    n_chunks = H // rows
    mesh = plsc.VectorSubcoreMesh(core_axis_name="c", subcore_axis_name="s")

    @functools.partial(
        pl.kernel,
        mesh=mesh,
        compiler_params=pltpu.CompilerParams(needs_layout_passes=False),
        out_type=jax.ShapeDtypeStruct((n_img, _L * 256), jnp.float32),
        scratch_types=[
            pltpu.VMEM((rows, W), jnp.int32),
            pltpu.VMEM((rows, W), jnp.int32),
            pltpu.VMEM((_L * 256,), jnp.float32),
            pltpu.SemaphoreType.DMA,
            pltpu.SemaphoreType.DMA,
        ],
    )
    def sc_hist(codes_hbm, out_hbm, buf0, buf1, hist_v, sem0, sem1):
        wid = lax.axis_index("s") * _NC + lax.axis_index("c")

        def zero_step(i, _):
            hist_v[pl.ds(i * _L, _L)] = jnp.zeros((_L,), jnp.float32)
            return 0
        lax.fori_loop(0, (_L * 256) // _L, zero_step, 0)

        lane256 = lax.broadcasted_iota(jnp.int32, (_L,), 0) * 256
        ones = jnp.ones((_L,), jnp.float32)

        bufs = [buf0, buf1]
        sems = [sem0, sem1]

        def start(c, slot):
            return pltpu.async_copy(
                codes_hbm.at[wid, pl.ds(c * rows, rows), :],
                bufs[slot], sems[slot])

        def consume(slot):
            buf = bufs[slot]

            def row_step(i, _):
                for j in range(W // _L):
                    c = buf[i, pl.ds(j * _L, _L)]
                    plsc.addupdate_scatter(hist_v, [c + lane256], ones)
                return 0
            lax.fori_loop(0, rows, row_step, 0)

        cp = start(0, 0)
        for c in range(n_chunks):
            slot = c % 2
            cp.wait()
            if c + 1 < n_chunks:
                nxt = start(c + 1, 1 - slot)
            consume(slot)
            if c + 1 < n_chunks:
                cp = nxt

        pltpu.sync_copy(hist_v, out_hbm.at[wid])

    return sc_hist


def _reduce_body(h_ref, o_ref):
    h = h_ref[0]
    s = jnp.sum(h, axis=0)
    norm = jnp.sqrt(jnp.sum(s * s))
    o_ref[0, 0] = s / (norm + 1e-6)


def kernel(x):
    bs, _, H, W = x.shape
    n_pix = H * W

    codes = pl.pallas_call(
        functools.partial(_code_body, H=H, W=W),
        grid=(bs,),
        in_specs=[pl.BlockSpec((1, 3, H, W), lambda i: (i, 0, 0, 0))],
        out_specs=pl.BlockSpec((1, H, W), lambda i: (i, 0, 0)),
        out_shape=jax.ShapeDtypeStruct((bs, H, W), jnp.int32),
    )(x)

    rows = 64 if H % 64 == 0 else H
    sc_hist = _make_sc_hist(bs, H, W, rows)
    hist_lanes = sc_hist(codes)

    out = pl.pallas_call(
        _reduce_body,
        grid=(bs,),
        in_specs=[pl.BlockSpec((1, _L, 256), lambda i: (i, 0, 0))],
        out_specs=pl.BlockSpec((1, 1, 256), lambda i: (i, 0, 0)),
        out_shape=jax.ShapeDtypeStruct((bs, 1, 256), jnp.float32),
    )(hist_lanes.reshape(bs, _L, 256))
    return out.reshape(bs, 256)


# speedup vs baseline: 34.9512x; 1.0578x over previous
import functools

import jax
import jax.numpy as jnp
from jax import lax
from jax.experimental import pallas as pl
from jax.experimental.pallas import tpu as pltpu
from jax.experimental.pallas import tpu_sc as plsc

_OFFSETS = [(-1, -1), (-1, 0), (-1, 1), (0, 1), (1, 1), (1, 0), (1, -1), (0, -1)]

_NC = 2
_NS = 16
_L = 16


def _code_body(x_ref, o_ref, *, H, W):
    r = x_ref[0, 0]
    g = x_ref[0, 1]
    b = x_ref[0, 2]
    gray = 0.2989 * r + 0.587 * g + 0.114 * b

    gp = jnp.concatenate([gray[:1], gray, gray[-1:]], axis=0)
    gp = jnp.concatenate([gp[:, :1], gp, gp[:, -1:]], axis=1)

    code = jnp.zeros((H, W), jnp.int32)
    for k, (dy, dx) in enumerate(_OFFSETS):
        neigh = jax.lax.slice(gp, (1 + dy, 1 + dx), (1 + dy + H, 1 + dx + W))
        code = code + (1 << k) * (neigh >= gray).astype(jnp.int32)
    o_ref[0] = code


def _make_sc_hist(n_img, H, W, rows):
    n_chunks = H // rows
    mesh = plsc.VectorSubcoreMesh(core_axis_name="c", subcore_axis_name="s")

    @functools.partial(
        pl.kernel,
        mesh=mesh,
        compiler_params=pltpu.CompilerParams(needs_layout_passes=False),
        out_type=jax.ShapeDtypeStruct((n_img, 256), jnp.float32),
        scratch_types=[
            pltpu.VMEM((rows, W), jnp.int32),
            pltpu.VMEM((rows, W), jnp.int32),
            pltpu.VMEM((_L * 256,), jnp.float32),
            pltpu.VMEM((256,), jnp.float32),
            pltpu.SemaphoreType.DMA,
            pltpu.SemaphoreType.DMA,
        ],
    )
    def sc_hist(codes_hbm, out_hbm, buf0, buf1, hist_v, out_v, sem0, sem1):
        wid = lax.axis_index("s") * _NC + lax.axis_index("c")

        def zero_step(i, _):
            hist_v[pl.ds(i * _L, _L)] = jnp.zeros((_L,), jnp.float32)
            return 0
        lax.fori_loop(0, (_L * 256) // _L, zero_step, 0)

        lane = lax.broadcasted_iota(jnp.int32, (_L,), 0)
        ones = jnp.ones((_L,), jnp.float32)

        bufs = [buf0, buf1]
        sems = [sem0, sem1]

        def start(c, slot):
            return pltpu.async_copy(
                codes_hbm.at[wid, pl.ds(c * rows, rows), :],
                bufs[slot], sems[slot])

        def consume(slot):
            buf = bufs[slot]

            def row_step(i, _):
                for j in range(W // _L):
                    c = buf[i, pl.ds(j * _L, _L)]
                    plsc.addupdate_scatter(hist_v, [c * _L + lane], ones)
                return 0
            lax.fori_loop(0, rows, row_step, 0)

        cp = start(0, 0)
        for c in range(n_chunks):
            slot = c % 2
            cp.wait()
            if c + 1 < n_chunks:
                nxt = start(c + 1, 1 - slot)
            consume(slot)
            if c + 1 < n_chunks:
                cp = nxt

        def fold_step(t, _):
            base = (t * _L + lane) * _L
            acc = jnp.zeros((_L,), jnp.float32)
            for l in range(_L):
                acc = acc + plsc.load_gather(hist_v, [base + l])
            out_v[pl.ds(t * _L, _L)] = acc
            return 0
        lax.fori_loop(0, 256 // _L, fold_step, 0)

        pltpu.sync_copy(out_v, out_hbm.at[wid])

    return sc_hist


def _reduce_body(h_ref, o_ref):
    s = h_ref[0]
    norm = jnp.sqrt(jnp.sum(s * s))
    o_ref[0] = s / (norm + 1e-6)


def kernel(x):
    bs, _, H, W = x.shape
    n_pix = H * W

    codes = pl.pallas_call(
        functools.partial(_code_body, H=H, W=W),
        grid=(bs,),
        in_specs=[pl.BlockSpec((1, 3, H, W), lambda i: (i, 0, 0, 0))],
        out_specs=pl.BlockSpec((1, H, W), lambda i: (i, 0, 0)),
        out_shape=jax.ShapeDtypeStruct((bs, H, W), jnp.int32),
    )(x)

    rows = 64 if H % 64 == 0 else H
    sc_hist = _make_sc_hist(bs, H, W, rows)
    hist = sc_hist(codes)

    out = pl.pallas_call(
        _reduce_body,
        grid=(bs,),
        in_specs=[pl.BlockSpec((1, 1, 256), lambda i: (i, 0, 0))],
        out_specs=pl.BlockSpec((1, 1, 256), lambda i: (i, 0, 0)),
        out_shape=jax.ShapeDtypeStruct((bs, 1, 256), jnp.float32),
    )(hist.reshape(bs, 1, 256))
    return out.reshape(bs, 256)
